# Initial kernel scaffold; baseline (speedup 1.0000x reference)
import jax
import jax.numpy as jnp
from jax.experimental import pallas as pl


def kernel(inputs, token_table, pos_table):
    raise NotImplementedError("write your pallas kernel here")


# speedup vs baseline: 3.3365x; 3.3365x over previous
import functools

import jax
import jax.numpy as jnp
from jax import lax
from jax.experimental import pallas as pl
from jax.experimental.pallas import tpu as pltpu
from jax.experimental.pallas import tpu_sc as plsc

LANES = 16


@functools.lru_cache(maxsize=None)
def _build(n_rows: int, vocab: int, embed: int, seq_len: int, chunk: int):
    info = plsc.get_sparse_core_info()
    nw = info.num_cores * info.num_subcores
    assert n_rows % nw == 0
    rows_per_w = n_rows // nw
    assert rows_per_w % chunk == 0
    n_chunks = rows_per_w // chunk
    assert chunk % seq_len == 0
    pos_rep = chunk // seq_len
    vecs_per_row = embed // LANES

    mesh = plsc.VectorSubcoreMesh(core_axis_name="c", subcore_axis_name="s")

    @functools.partial(
        pl.kernel,
        out_type=jax.ShapeDtypeStruct((n_rows, embed), jnp.float32),
        mesh=mesh,
        scratch_types=[
            pltpu.VMEM((chunk,), jnp.int32),
            pltpu.VMEM((chunk, embed), jnp.float32),
            pltpu.VMEM((chunk, embed), jnp.float32),
            pltpu.SemaphoreType.DMA,
        ],
        compiler_params=pltpu.CompilerParams(use_tc_tiling_on_sc=False),
    )
    def emb_kernel(table_hbm, idx_hbm, pos_hbm, out_hbm, idx_v, rows_v, pos_v, sem):
        wid = lax.axis_index("s") * info.num_cores + lax.axis_index("c")
        base = wid * rows_per_w

        for r in range(pos_rep):
            pltpu.sync_copy(pos_hbm, pos_v.at[pl.ds(r * seq_len, seq_len)])

        def chunk_body(c, _):
            off = base + c * chunk
            pltpu.sync_copy(idx_hbm.at[pl.ds(off, chunk)], idx_v)
            pltpu.async_copy(table_hbm.at[idx_v], rows_v, sem).wait()

            def add_body(s, _):
                for k in range(vecs_per_row):
                    ds = pl.ds(k * LANES, LANES)
                    rows_v[s, ds] = rows_v[s, ds] + pos_v[s, ds]
                return _

            lax.fori_loop(0, chunk, add_body, None)
            pltpu.sync_copy(rows_v, out_hbm.at[pl.ds(off, chunk)])
            return _

        lax.fori_loop(0, n_chunks, chunk_body, None)

    return emb_kernel


def kernel(inputs, token_table, pos_table):
    batch, seq_len = inputs.shape
    vocab, embed = token_table.shape
    flat_idx = inputs.reshape(-1).astype(jnp.int32)
    fn = _build(batch * seq_len, vocab, embed, seq_len, chunk=2 * seq_len)
    out = fn(token_table, flat_idx, pos_table)
    return out.reshape(batch, seq_len, embed)
